# Initial kernel scaffold; baseline (speedup 1.0000x reference)
import jax
import jax.numpy as jnp
from jax.experimental import pallas as pl


def kernel(x, edge_index, W1l, W1r, b1, g1, be1, W2l, W2r, b2, g2, be2, W3l, W3r, b3):
    raise NotImplementedError("write your pallas kernel here")


# speedup vs baseline: 5.6068x; 5.6068x over previous
import functools

import jax
import jax.numpy as jnp
from jax import lax
from jax.experimental import pallas as pl
from jax.experimental.pallas import tpu as pltpu
from jax.experimental.pallas import tpu_sc as plsc

N = 10000
E = 320000
D = 128
H = 128
O = 64

NC = 2
NS = 16
NW = NC * NS
EPT = E // NW
K = 80
NCHUNK = EPT // K
ROWS_PER_TILE = 624
TAIL_ROWS = N - NS * ROWS_PER_TILE


def _sc_aggregate(scatter_to_src: bool, width: int):
    mesh = plsc.VectorSubcoreMesh(core_axis_name="c", subcore_axis_name="s")

    def body(feat_hbm, src_hbm, dst_hbm, wtab_hbm, zrows_hbm, zvec_hbm,
             acc_out, side_out,
             acc_sh, side_sh, sidx, didx, rows, wvec, sem):
        c = lax.axis_index("c")
        s = lax.axis_index("s")
        base = (c * NS + s) * EPT

        r0 = s * ROWS_PER_TILE
        pltpu.sync_copy(zrows_hbm.at[pl.ds(r0, ROWS_PER_TILE)],
                        acc_sh.at[pl.ds(r0, ROWS_PER_TILE)])

        @pl.when(s == 0)
        def _():
            tail = NS * ROWS_PER_TILE
            pltpu.sync_copy(zrows_hbm.at[pl.ds(tail, TAIL_ROWS)],
                            acc_sh.at[pl.ds(tail, TAIL_ROWS)])
            pltpu.sync_copy(zvec_hbm, side_sh)

        plsc.subcore_barrier()

        def chunk(i, carry):
            off = base + i * K
            pltpu.sync_copy(src_hbm.at[pl.ds(off, K)], sidx)
            pltpu.sync_copy(dst_hbm.at[pl.ds(off, K)], didx)
            pltpu.async_copy(feat_hbm.at[sidx], rows, sem).wait()
            pltpu.sync_copy(rows, acc_sh.at[didx], add=True)
            pltpu.async_copy(wtab_hbm.at[didx], wvec, sem).wait()
            if scatter_to_src:
                pltpu.sync_copy(wvec, side_sh.at[sidx], add=True)
            else:
                pltpu.sync_copy(wvec, side_sh.at[didx], add=True)
            return carry

        lax.fori_loop(0, NCHUNK, chunk, 0)

        plsc.subcore_barrier()

        pltpu.sync_copy(acc_sh.at[pl.ds(r0, ROWS_PER_TILE)],
                        acc_out.at[c, pl.ds(r0, ROWS_PER_TILE)])

        @pl.when(s == 0)
        def _():
            tail = NS * ROWS_PER_TILE
            pltpu.sync_copy(acc_sh.at[pl.ds(tail, TAIL_ROWS)],
                            acc_out.at[c, pl.ds(tail, TAIL_ROWS)])
            pltpu.sync_copy(side_sh, side_out.at[c])

    return pl.kernel(
        body,
        out_type=[
            jax.ShapeDtypeStruct((NC, N, width), jnp.float32),
            jax.ShapeDtypeStruct((NC, N), jnp.float32),
        ],
        mesh=mesh,
        scratch_types=[
            pltpu.VMEM_SHARED((N, width), jnp.float32),
            pltpu.VMEM_SHARED((N,), jnp.float32),
            pltpu.VMEM((K,), jnp.int32),
            pltpu.VMEM((K,), jnp.int32),
            pltpu.VMEM((K, width), jnp.float32),
            pltpu.VMEM((K,), jnp.float32),
            pltpu.SemaphoreType.DMA,
        ],
    )


def _layer1_body(acc_ref, cnt_ref, x_ref, wl_ref, wr_ref, b_ref, g_ref,
                 be_ref, h_ref, inv_ref):
    cnt = cnt_ref[0] + cnt_ref[1]
    inv = 1.0 / jnp.maximum(cnt, 1.0)
    agg = acc_ref[0] + acc_ref[1]
    mean = agg * inv[:, None]
    h = (jnp.dot(mean, wl_ref[...], preferred_element_type=jnp.float32)
         + jnp.dot(x_ref[...], wr_ref[...], preferred_element_type=jnp.float32)
         + b_ref[...])
    mu = jnp.mean(h, axis=0)
    var = jnp.mean((h - mu) ** 2, axis=0)
    hn = g_ref[...] * (h - mu) / jnp.sqrt(var + 1e-5) + be_ref[...]
    h_ref[...] = jnp.maximum(hn, 0.0)
    inv_ref[...] = inv


def _layer23_body(acc_ref, s_ref, inv_ref, h1_ref, wl_ref, wr_ref, b_ref,
                  g_ref, be_ref, w3l_ref, w3r_ref, b3_ref, out_ref):
    inv = inv_ref[...]
    agg = acc_ref[0] + acc_ref[1]
    mean = agg * inv[:, None]
    h = (jnp.dot(mean, wl_ref[...], preferred_element_type=jnp.float32)
         + jnp.dot(h1_ref[...], wr_ref[...], preferred_element_type=jnp.float32)
         + b_ref[...])
    mu = jnp.mean(h, axis=0)
    var = jnp.mean((h - mu) ** 2, axis=0)
    hn = g_ref[...] * (h - mu) / jnp.sqrt(var + 1e-5) + be_ref[...]
    h2 = jnp.maximum(hn, 0.0)
    ssum = s_ref[0] + s_ref[1]
    v1 = jnp.sum(h2 * ssum[:, None], axis=0, keepdims=True)
    v0 = jnp.sum(h2, axis=0, keepdims=True)
    state = (jnp.dot(v1, w3l_ref[...], preferred_element_type=jnp.float32)
             + jnp.dot(v0, w3r_ref[...], preferred_element_type=jnp.float32)
             ) * (1.0 / N) + b3_ref[...][None, :]
    out_ref[...] = state


def kernel(x, edge_index, W1l, W1r, b1, g1, be1, W2l, W2r, b2, g2, be2,
           W3l, W3r, b3):
    src = edge_index[0].astype(jnp.int32)
    dst = edge_index[1].astype(jnp.int32)
    zrows = jnp.zeros((N, D), jnp.float32)
    zvec = jnp.zeros((N,), jnp.float32)
    ones_n = jnp.ones((N,), jnp.float32)

    agg1, cnt = _sc_aggregate(False, D)(x, src, dst, ones_n, zrows, zvec)
    h1, inv = pl.pallas_call(
        _layer1_body,
        out_shape=[
            jax.ShapeDtypeStruct((N, H), jnp.float32),
            jax.ShapeDtypeStruct((N,), jnp.float32),
        ],
    )(agg1, cnt, x, W1l, W1r, b1, g1, be1)

    agg2, svec = _sc_aggregate(True, H)(h1, src, dst, inv, zrows, zvec)
    state = pl.pallas_call(
        _layer23_body,
        out_shape=jax.ShapeDtypeStruct((1, O), jnp.float32),
    )(agg2, svec, inv, h1, W2l, W2r, b2, g2, be2, W3l, W3r, b3)
    return state.reshape(O)
